# baseline (speedup 1.0000x reference)
import functools

import jax
import jax.numpy as jnp
from jax import lax
from jax.experimental import pallas as pl
from jax.experimental.pallas import tpu as pltpu
from jax.experimental.pallas import tpu_sc as plsc

NUM_CLASSES = 100000
FEATURE_DIM = 128
BATCH = 16384

_NC = 2
_NS = 16
_NW = _NC * _NS
_CHUNK = 128
_NCHUNK = BATCH // (_NW * _CHUNK)


def _gather_kernel(centers_hbm, idx_hbm, out_hbm, idx_v, rows_v, gsems, wsems):
    wid = lax.axis_index("s") * _NC + lax.axis_index("c")
    base = wid * _NCHUNK
    pltpu.sync_copy(idx_hbm.at[pl.ds(base, _NCHUNK)], idx_v)
    gathers = [
        pltpu.async_copy(centers_hbm.at[idx_v.at[j]], rows_v.at[j], gsems.at[j])
        for j in range(_NCHUNK)
    ]
    writes = []
    for j in range(_NCHUNK):
        gathers[j].wait()
        writes.append(
            pltpu.async_copy(rows_v.at[j], out_hbm.at[base + j], wsems.at[j])
        )
    for c in writes:
        c.wait()


@jax.jit
def kernel(labels, centers):
    idx2d = labels.astype(jnp.int32).reshape(_NW * _NCHUNK, _CHUNK)
    mesh = plsc.VectorSubcoreMesh(core_axis_name="c", subcore_axis_name="s")
    out3d = pl.kernel(
        _gather_kernel,
        mesh=mesh,
        out_type=jax.ShapeDtypeStruct((_NW * _NCHUNK, _CHUNK, FEATURE_DIM), jnp.float32),
        scratch_types=[
            pltpu.VMEM((_NCHUNK, _CHUNK), jnp.int32),
            pltpu.VMEM((_NCHUNK, _CHUNK, FEATURE_DIM), jnp.float32),
            pltpu.SemaphoreType.DMA((_NCHUNK,)),
            pltpu.SemaphoreType.DMA((_NCHUNK,)),
        ],
    )(centers, idx2d)
    return out3d.reshape(BATCH, FEATURE_DIM)


# speedup vs baseline: 1.0091x; 1.0091x over previous
import functools

import jax
import jax.numpy as jnp
from jax import lax
from jax.experimental import pallas as pl
from jax.experimental.pallas import tpu as pltpu
from jax.experimental.pallas import tpu_sc as plsc

NUM_CLASSES = 100000
FEATURE_DIM = 128
BATCH = 16384

_NC = 2
_NS = 16
_NW = _NC * _NS
_BPW = BATCH // _NW


def _gather_kernel(centers_hbm, idx_hbm, out_hbm, idx_v, rows_v, sem):
    wid = lax.axis_index("s") * _NC + lax.axis_index("c")
    pltpu.sync_copy(idx_hbm.at[wid], idx_v)
    pltpu.async_copy(centers_hbm.at[idx_v], rows_v, sem).wait()
    pltpu.sync_copy(rows_v, out_hbm.at[wid])


@jax.jit
def kernel(labels, centers):
    idx2d = labels.astype(jnp.int32).reshape(_NW, _BPW)
    mesh = plsc.VectorSubcoreMesh(core_axis_name="c", subcore_axis_name="s")
    out3d = pl.kernel(
        _gather_kernel,
        mesh=mesh,
        out_type=jax.ShapeDtypeStruct((_NW, _BPW, FEATURE_DIM), jnp.float32),
        scratch_types=[
            pltpu.VMEM((_BPW,), jnp.int32),
            pltpu.VMEM((_BPW, FEATURE_DIM), jnp.float32),
            pltpu.SemaphoreType.DMA,
        ],
    )(centers, idx2d)
    return out3d.reshape(BATCH, FEATURE_DIM)


# speedup vs baseline: 1.0178x; 1.0086x over previous
import functools

import jax
import jax.numpy as jnp
from jax import lax
from jax.experimental import pallas as pl
from jax.experimental.pallas import tpu as pltpu
from jax.experimental.pallas import tpu_sc as plsc

NUM_CLASSES = 100000
FEATURE_DIM = 128
BATCH = 16384

_NC = 2
_NS = 16
_NW = _NC * _NS
_BPW = BATCH // _NW


def _gather_kernel(centers_hbm, idx_hbm, out_hbm, idx_v, rows_v, sem):
    wid = lax.axis_index("s") * _NC + lax.axis_index("c")
    base = wid * _BPW
    pltpu.sync_copy(idx_hbm.at[pl.ds(base, _BPW)], idx_v)
    pltpu.async_copy(centers_hbm.at[idx_v], rows_v, sem).wait()
    pltpu.sync_copy(rows_v, out_hbm.at[pl.ds(base, _BPW)])


@jax.jit
def kernel(labels, centers):
    idx = labels.astype(jnp.int32)
    mesh = plsc.VectorSubcoreMesh(core_axis_name="c", subcore_axis_name="s")
    return pl.kernel(
        _gather_kernel,
        mesh=mesh,
        out_type=jax.ShapeDtypeStruct((BATCH, FEATURE_DIM), jnp.float32),
        scratch_types=[
            pltpu.VMEM((_BPW,), jnp.int32),
            pltpu.VMEM((_BPW, FEATURE_DIM), jnp.float32),
            pltpu.SemaphoreType.DMA,
        ],
    )(centers, idx)
